# initial kernel scaffold (unmeasured)
import jax
import jax.numpy as jnp
from jax import lax
from jax.experimental import pallas as pl
from jax.experimental.pallas import tpu as pltpu

MC = 256
NSLOTS = 4


def kernel(A, B):
    M, Kl = A.shape
    K2, N = B.shape
    assert Kl == K2
    C = M // MC

    def body(a_hbm, b_hbm, o_hbm, b_vmem, a_vmem, p_vmem, recv_vmem,
             b_sem, a_sem, store_sem, send_sem, recv_sems):
        my_x = lax.axis_index("x")
        my_y = lax.axis_index("y")
        nbr = (1 - my_x, my_y)

        barrier = pltpu.get_barrier_semaphore()
        pl.semaphore_signal(
            barrier, inc=1, device_id=nbr,
            device_id_type=pl.DeviceIdType.MESH,
        )
        pl.semaphore_wait(barrier, 1)

        b_load = pltpu.make_async_copy(b_hbm, b_vmem, b_sem)
        b_load.start()
        b_load.wait()

        for c in range(C):
            slot = c % NSLOTS
            a_load = pltpu.make_async_copy(
                a_hbm.at[pl.ds(c * MC, MC)], a_vmem, a_sem)
            a_load.start()
            a_load.wait()
            p_vmem[...] = jnp.dot(
                a_vmem[...], b_vmem[...],
                preferred_element_type=jnp.float32)
            rdma = pltpu.make_async_remote_copy(
                src_ref=p_vmem,
                dst_ref=recv_vmem.at[slot],
                send_sem=send_sem,
                recv_sem=recv_sems.at[slot],
                device_id=nbr,
                device_id_type=pl.DeviceIdType.MESH,
            )
            rdma.start()
            rdma.wait()
            p_vmem[...] = p_vmem[...] + recv_vmem[slot]
            store = pltpu.make_async_copy(
                p_vmem, o_hbm.at[pl.ds(c * MC, MC)], store_sem)
            store.start()
            store.wait()

    return pl.pallas_call(
        body,
        out_shape=jax.ShapeDtypeStruct((M, N), jnp.float32),
        in_specs=[
            pl.BlockSpec(memory_space=pltpu.ANY),
            pl.BlockSpec(memory_space=pltpu.ANY),
        ],
        out_specs=pl.BlockSpec(memory_space=pltpu.ANY),
        scratch_shapes=[
            pltpu.VMEM((K2, N), jnp.float32),
            pltpu.VMEM((MC, Kl), jnp.float32),
            pltpu.VMEM((MC, N), jnp.float32),
            pltpu.VMEM((NSLOTS, MC, N), jnp.float32),
            pltpu.SemaphoreType.DMA,
            pltpu.SemaphoreType.DMA,
            pltpu.SemaphoreType.DMA,
            pltpu.SemaphoreType.DMA,
            pltpu.SemaphoreType.DMA((NSLOTS,)),
        ],
        compiler_params=pltpu.CompilerParams(collective_id=0),
    )(A, B)


# baseline (device time: 945514 ns/iter reference)
import jax
import jax.numpy as jnp
from jax import lax
from jax.experimental import pallas as pl
from jax.experimental.pallas import tpu as pltpu

MC = 256
NSLOTS = 4


def kernel(A, B):
    M, Kl = A.shape
    K2, N = B.shape
    assert Kl == K2
    C = M // MC

    def body(a_hbm, b_hbm, o_hbm, b_vmem, a_vmem, p_vmem, recv_vmem,
             b_sem, a_sem, store_sem, send_sem, recv_sems):
        my_x = lax.axis_index("x")
        my_y = lax.axis_index("y")
        nbr = (1 - my_x, my_y)

        barrier = pltpu.get_barrier_semaphore()
        pl.semaphore_signal(
            barrier, inc=1, device_id=nbr,
            device_id_type=pl.DeviceIdType.MESH,
        )
        pl.semaphore_wait(barrier, 1)

        b_load = pltpu.make_async_copy(b_hbm, b_vmem, b_sem)
        b_load.start()
        b_load.wait()

        def step(c, carry):
            slot = lax.rem(c, NSLOTS)
            a_load = pltpu.make_async_copy(
                a_hbm.at[pl.ds(c * MC, MC)], a_vmem, a_sem)
            a_load.start()
            a_load.wait()
            p_vmem[...] = jnp.dot(
                a_vmem[...], b_vmem[...],
                preferred_element_type=jnp.float32)
            rdma = pltpu.make_async_remote_copy(
                src_ref=p_vmem,
                dst_ref=recv_vmem.at[slot],
                send_sem=send_sem,
                recv_sem=recv_sems.at[slot],
                device_id=nbr,
                device_id_type=pl.DeviceIdType.MESH,
            )
            rdma.start()
            rdma.wait()
            p_vmem[...] = p_vmem[...] + recv_vmem[slot]
            store = pltpu.make_async_copy(
                p_vmem, o_hbm.at[pl.ds(c * MC, MC)], store_sem)
            store.start()
            store.wait()
            return carry

        lax.fori_loop(0, C, step, 0)

    return pl.pallas_call(
        body,
        out_shape=jax.ShapeDtypeStruct((M, N), jnp.float32),
        in_specs=[
            pl.BlockSpec(memory_space=pl.ANY),
            pl.BlockSpec(memory_space=pl.ANY),
        ],
        out_specs=pl.BlockSpec(memory_space=pl.ANY),
        scratch_shapes=[
            pltpu.VMEM((K2, N), jnp.float32),
            pltpu.VMEM((MC, Kl), jnp.float32),
            pltpu.VMEM((MC, N), jnp.float32),
            pltpu.VMEM((NSLOTS, MC, N), jnp.float32),
            pltpu.SemaphoreType.DMA,
            pltpu.SemaphoreType.DMA,
            pltpu.SemaphoreType.DMA,
            pltpu.SemaphoreType.DMA,
            pltpu.SemaphoreType.DMA((NSLOTS,)),
        ],
        compiler_params=pltpu.CompilerParams(
            collective_id=0,
            vmem_limit_bytes=60 * 1024 * 1024,
        ),
    )(A, B)


# device time: 785083 ns/iter; 1.2043x vs baseline; 1.2043x over previous
import jax
import jax.numpy as jnp
from jax import lax
from jax.experimental import pallas as pl
from jax.experimental.pallas import tpu as pltpu

MC = 256
NSLOTS = 4


def kernel(A, B):
    M, Kl = A.shape
    K2, N = B.shape
    assert Kl == K2
    C = M // MC

    def body(a_hbm, b_hbm, o_hbm, b_vmem, a_vmem, p_vmem, recv_vmem,
             b_sem, a_sems, store_sems, send_sems, recv_sems):
        my_x = lax.axis_index("x")
        my_y = lax.axis_index("y")
        nbr = (1 - my_x, my_y)

        def a_load(c):
            return pltpu.make_async_copy(
                a_hbm.at[pl.ds(c * MC, MC)],
                a_vmem.at[lax.rem(c, 2)],
                a_sems.at[lax.rem(c, 2)])

        def exchange(c):
            return pltpu.make_async_remote_copy(
                src_ref=p_vmem.at[lax.rem(c, 2)],
                dst_ref=recv_vmem.at[lax.rem(c, NSLOTS)],
                send_sem=send_sems.at[lax.rem(c, 2)],
                recv_sem=recv_sems.at[lax.rem(c, NSLOTS)],
                device_id=nbr,
                device_id_type=pl.DeviceIdType.MESH,
            )

        def store(c):
            return pltpu.make_async_copy(
                p_vmem.at[lax.rem(c, 2)],
                o_hbm.at[pl.ds(c * MC, MC)],
                store_sems.at[lax.rem(c, 2)])

        def consume(c):
            exchange(c).wait_recv()
            exchange(c).wait_send()
            slot = lax.rem(c, 2)
            p_vmem[slot] = p_vmem[slot] + recv_vmem[lax.rem(c, NSLOTS)]
            store(c).start()

        barrier = pltpu.get_barrier_semaphore()
        pl.semaphore_signal(
            barrier, inc=1, device_id=nbr,
            device_id_type=pl.DeviceIdType.MESH,
        )
        pl.semaphore_wait(barrier, 1)

        b_load = pltpu.make_async_copy(b_hbm, b_vmem, b_sem)
        b_load.start()
        a_load(0).start()
        b_load.wait()

        def step(c, carry):
            a_load(c).wait()

            @pl.when(c >= 2)
            def _():
                store(c - 2).wait()

            p_vmem[lax.rem(c, 2)] = jnp.dot(
                a_vmem[lax.rem(c, 2)], b_vmem[...],
                preferred_element_type=jnp.float32)
            exchange(c).start()

            @pl.when(c + 1 < C)
            def _():
                a_load(c + 1).start()

            @pl.when(c >= 1)
            def _():
                consume(c - 1)

            return carry

        lax.fori_loop(0, C, step, 0)

        store(C - 2).wait()
        consume(C - 1)
        store(C - 1).wait()

    return pl.pallas_call(
        body,
        out_shape=jax.ShapeDtypeStruct((M, N), jnp.float32),
        in_specs=[
            pl.BlockSpec(memory_space=pl.ANY),
            pl.BlockSpec(memory_space=pl.ANY),
        ],
        out_specs=pl.BlockSpec(memory_space=pl.ANY),
        scratch_shapes=[
            pltpu.VMEM((K2, N), jnp.float32),
            pltpu.VMEM((2, MC, Kl), jnp.float32),
            pltpu.VMEM((2, MC, N), jnp.float32),
            pltpu.VMEM((NSLOTS, MC, N), jnp.float32),
            pltpu.SemaphoreType.DMA,
            pltpu.SemaphoreType.DMA((2,)),
            pltpu.SemaphoreType.DMA((2,)),
            pltpu.SemaphoreType.DMA((2,)),
            pltpu.SemaphoreType.DMA((NSLOTS,)),
        ],
        compiler_params=pltpu.CompilerParams(
            collective_id=0,
            vmem_limit_bytes=62 * 1024 * 1024,
        ),
    )(A, B)


# device time: 781851 ns/iter; 1.2093x vs baseline; 1.0041x over previous
import jax
import jax.numpy as jnp
from jax import lax
from jax.experimental import pallas as pl
from jax.experimental.pallas import tpu as pltpu

MC = 128
NSLOTS = 4


def kernel(A, B):
    M, Kl = A.shape
    K2, N = B.shape
    assert Kl == K2
    C = M // MC

    def body(a_hbm, b_hbm, o_hbm, b_vmem, a_vmem, p_vmem, recv_vmem,
             b_sem, a_sems, store_sems, send_sems, recv_sems):
        my_x = lax.axis_index("x")
        my_y = lax.axis_index("y")
        nbr = (1 - my_x, my_y)

        def a_load(c):
            return pltpu.make_async_copy(
                a_hbm.at[pl.ds(c * MC, MC)],
                a_vmem.at[lax.rem(c, 2)],
                a_sems.at[lax.rem(c, 2)])

        def exchange(c):
            return pltpu.make_async_remote_copy(
                src_ref=p_vmem.at[lax.rem(c, 2)],
                dst_ref=recv_vmem.at[lax.rem(c, NSLOTS)],
                send_sem=send_sems.at[lax.rem(c, 2)],
                recv_sem=recv_sems.at[lax.rem(c, NSLOTS)],
                device_id=nbr,
                device_id_type=pl.DeviceIdType.MESH,
            )

        def store(c):
            return pltpu.make_async_copy(
                p_vmem.at[lax.rem(c, 2)],
                o_hbm.at[pl.ds(c * MC, MC)],
                store_sems.at[lax.rem(c, 2)])

        def consume(c):
            exchange(c).wait_recv()
            exchange(c).wait_send()
            slot = lax.rem(c, 2)
            p_vmem[slot] = p_vmem[slot] + recv_vmem[lax.rem(c, NSLOTS)]
            store(c).start()

        barrier = pltpu.get_barrier_semaphore()
        pl.semaphore_signal(
            barrier, inc=1, device_id=nbr,
            device_id_type=pl.DeviceIdType.MESH,
        )
        pl.semaphore_wait(barrier, 1)

        b_load = pltpu.make_async_copy(b_hbm, b_vmem, b_sem)
        b_load.start()
        a_load(0).start()
        b_load.wait()

        def step(c, carry):
            a_load(c).wait()

            @pl.when(c >= 2)
            def _():
                store(c - 2).wait()

            p_vmem[lax.rem(c, 2)] = jnp.dot(
                a_vmem[lax.rem(c, 2)], b_vmem[...],
                preferred_element_type=jnp.float32)
            exchange(c).start()

            @pl.when(c + 1 < C)
            def _():
                a_load(c + 1).start()

            @pl.when(c >= 1)
            def _():
                consume(c - 1)

            return carry

        lax.fori_loop(0, C, step, 0)

        store(C - 2).wait()
        consume(C - 1)
        store(C - 1).wait()

    return pl.pallas_call(
        body,
        out_shape=jax.ShapeDtypeStruct((M, N), jnp.float32),
        in_specs=[
            pl.BlockSpec(memory_space=pl.ANY),
            pl.BlockSpec(memory_space=pl.ANY),
        ],
        out_specs=pl.BlockSpec(memory_space=pl.ANY),
        scratch_shapes=[
            pltpu.VMEM((K2, N), jnp.float32),
            pltpu.VMEM((2, MC, Kl), jnp.float32),
            pltpu.VMEM((2, MC, N), jnp.float32),
            pltpu.VMEM((NSLOTS, MC, N), jnp.float32),
            pltpu.SemaphoreType.DMA,
            pltpu.SemaphoreType.DMA((2,)),
            pltpu.SemaphoreType.DMA((2,)),
            pltpu.SemaphoreType.DMA((2,)),
            pltpu.SemaphoreType.DMA((NSLOTS,)),
        ],
        compiler_params=pltpu.CompilerParams(
            collective_id=0,
            vmem_limit_bytes=62 * 1024 * 1024,
        ),
    )(A, B)
